# baseline (speedup 1.0000x reference)
import functools

import jax
import jax.numpy as jnp
from jax import lax
from jax.experimental import pallas as pl
from jax.experimental.pallas import tpu as pltpu
from jax.experimental.pallas import tpu_sc as plsc

KPTS = 100000
NC, NS, L = 2, 16, 16
NW = NC * NS
NB = 32
CSTRIDE = 1008
NJ = CSTRIDE // L
TBL = NB * CSTRIDE
CH = 4096
NCHUNK = 25
KP = 100352
KS = CH * NCHUNK
NVREG = CH // L
TAILV = (KPTS - (NCHUNK - 1) * CH) // L
MLP_BLK = 10240
NEG = -9999.0


def _mlp_body(w1, b1, w2, b2, w3, b3, w4, b4, x_ref, r_ref, c_ref,
              h_ref, gi_ref):
    x = x_ref[...]
    h = jnp.dot(w1[...], x, preferred_element_type=jnp.float32) + b1[...]
    h = jnp.maximum(h, 0.0)
    h = jnp.dot(w2[...], h, preferred_element_type=jnp.float32) + b2[...]
    h = jnp.maximum(h, 0.0)
    h = jnp.dot(w3[...], h, preferred_element_type=jnp.float32) + b3[...]
    h = jnp.maximum(h, 0.0)
    h_ref[...] = jnp.dot(w4[...], h, preferred_element_type=jnp.float32) + b4[...]
    gi_ref[...] = r_ref[...] * CSTRIDE + c_ref[...]


def _run_mlp(x_p, rr, cc, W1, b1, W2, b2, W3, b3, W4, b4):
    full = lambda shape: pl.BlockSpec(shape, lambda i: (0, 0))
    blk = pl.BlockSpec((1, MLP_BLK), lambda i: (0, i))
    return pl.pallas_call(
        _mlp_body,
        grid=(pl.cdiv(KP, MLP_BLK),),
        in_specs=[
            full((18, 9)), full((18, 1)),
            full((36, 18)), full((36, 1)),
            full((36, 36)), full((36, 1)),
            full((1, 36)), full((1, 1)),
            pl.BlockSpec((9, MLP_BLK), lambda i: (0, i)),
            blk, blk,
        ],
        out_specs=(blk, blk),
        out_shape=(jax.ShapeDtypeStruct((1, KP), jnp.float32),
                   jax.ShapeDtypeStruct((1, KP), jnp.int32)),
    )(W1, b1.reshape(18, 1), W2, b2.reshape(36, 1), W3, b3.reshape(36, 1),
      W4, b4.reshape(1, 1), x_p, rr.reshape(1, KPTS), cc.reshape(1, KPTS))


_sc_mesh = plsc.VectorSubcoreMesh(core_axis_name="c", subcore_axis_name="s")


@functools.partial(
    pl.kernel,
    out_type=(jax.ShapeDtypeStruct((NW * NB,), jnp.float32),
              jax.ShapeDtypeStruct((NW, CSTRIDE), jnp.float32)),
    mesh=_sc_mesh,
    scratch_types=[
        pltpu.VMEM((TBL,), jnp.float32),
        pltpu.VMEM((2, CH), jnp.int32),
        pltpu.VMEM((2, CH), jnp.float32),
        pltpu.VMEM((NB,), jnp.float32),
        pltpu.VMEM((CSTRIDE,), jnp.float32),
        pltpu.VMEM_SHARED((KS,), jnp.int32),
        pltpu.VMEM_SHARED((KS,), jnp.float32),
        pltpu.SemaphoreType.DMA,
        pltpu.SemaphoreType.DMA,
        pltpu.SemaphoreType.DMA,
    ],
    compiler_params=pltpu.CompilerParams(needs_layout_passes=False),
)
def _sc_scatter(gi_hbm, h_hbm, x1a_hbm, colp_hbm,
                tbl, gb, hb, rowo, colo, gis, hs, sem0, sem1, psem):
    cid = lax.axis_index("c")
    sid = lax.axis_index("s")
    wid = sid * NC + cid
    base = wid * NB
    gilo = base * CSTRIDE
    lanes = lax.iota(jnp.int32, L)
    sems = (sem0, sem1)
    negv = jnp.full((L,), NEG, jnp.float32)

    seg = KP // NS
    soff = sid * seg
    pltpu.async_copy(gi_hbm.at[pl.ds(soff, seg)], gis.at[pl.ds(soff, seg)], psem)
    pltpu.async_copy(h_hbm.at[pl.ds(soff, seg)], hs.at[pl.ds(soff, seg)], psem)

    @plsc.parallel_loop(0, TBL // L, unroll=8)
    def _(i):
        tbl[pl.ds(i * L, L)] = negv

    pltpu.make_async_copy(gi_hbm.at[pl.ds(soff, seg)], gis.at[pl.ds(soff, seg)], psem).wait()
    pltpu.make_async_copy(h_hbm.at[pl.ds(soff, seg)], hs.at[pl.ds(soff, seg)], psem).wait()
    plsc.subcore_barrier()

    def start(slot, ci):
        off = ci * CH
        pltpu.async_copy(gis.at[pl.ds(off, CH)], gb.at[slot], sems[slot])
        pltpu.async_copy(hs.at[pl.ds(off, CH)], hb.at[slot], sems[slot])

    def wait(slot):
        pltpu.make_async_copy(gis.at[pl.ds(0, CH)], gb.at[slot], sems[slot]).wait()
        pltpu.make_async_copy(hs.at[pl.ds(0, CH)], hb.at[slot], sems[slot]).wait()

    def process(slot, nvreg, unroll):
        def vbody(i, c):
            s = i * (unroll * L)
            for j in range(unroll):
                gv = gb[slot, pl.ds(s + j * L, L)]
                hv = hb[slot, pl.ds(s + j * L, L)]
                li = gv - gilo
                m = (li >= 0) & (li < TBL)
                li = jnp.where(m, li, 0)
                plsc.store_scatter(tbl, [li], hv, mask=m)
            return c
        lax.fori_loop(0, nvreg // unroll, vbody, 0)

    start(0, 0)

    def pair_body(cj, c):
        ci0 = cj * 2
        start(1, ci0 + 1)
        wait(0)
        process(0, NVREG, 8)
        start(0, ci0 + 2)
        wait(1)
        process(1, NVREG, 8)
        return c
    lax.fori_loop(0, (NCHUNK - 1) // 2, pair_body, 0)
    wait(0)
    process(0, TAILV, 2)

    for g in range(NB // L):
        @plsc.parallel_loop(0, L, carry=negv)
        def rowvec(k, rowvec):
            rl = g * L + k
            acc = negv
            for j in range(NJ):
                acc = jnp.maximum(acc, tbl[pl.ds(rl * CSTRIDE + j * L, L)])
            rmax = jnp.max(acc)
            rmax = jnp.where(base + rl < 1000, rmax, NEG)
            return jnp.where(lanes == k, rmax, rowvec)
        rowo[pl.ds(g * L, L)] = rowvec

    nvalid = jnp.maximum(jnp.minimum(NB, 1000 - base), 0)

    for jb in range(7):
        @plsc.parallel_loop(0, nvalid, carry=(negv,) * 9)
        def accs(rl, accs):
            rbase = rl * CSTRIDE + jb * 9 * L
            return tuple(
                jnp.maximum(accs[u], tbl[pl.ds(rbase + u * L, L)])
                for u in range(9))
        for u in range(9):
            colo[pl.ds((jb * 9 + u) * L, L)] = accs[u]

    pltpu.sync_copy(rowo, x1a_hbm.at[pl.ds(base, NB)])
    pltpu.sync_copy(colo, colp_hbm.at[wid])


def _combine_body(x1a_ref, colp_ref, x1_ref, x2_ref):
    x1_ref[0, 0:NW * NB] = x1a_ref[0, :]
    x1_ref[0, NW * NB:2000] = jnp.full((2000 - NW * NB,), NEG, jnp.float32)
    red = jnp.max(colp_ref[...], axis=0)
    x2_ref[0, :] = red[0:1000]


def _run_combine(x1a, colp):
    return pl.pallas_call(
        _combine_body,
        in_specs=[pl.BlockSpec((1, NW * NB), lambda: (0, 0)),
                  pl.BlockSpec((NW, CSTRIDE), lambda: (0, 0))],
        out_specs=(pl.BlockSpec((1, 2000), lambda: (0, 0)),
                   pl.BlockSpec((1, 1000), lambda: (0, 0))),
        out_shape=(jax.ShapeDtypeStruct((1, 2000), jnp.float32),
                   jax.ShapeDtypeStruct((1, 1000), jnp.float32)),
    )(x1a.reshape(1, NW * NB), colp)


def kernel(input, T_out, T_indices, W1, b1, W2, b2, W3, b3, W4, b4):
    x = input[0, :, 0, :]
    h, gi = _run_mlp(x, T_indices[0], T_indices[1],
                     W1, b1, W2, b2, W3, b3, W4, b4)
    x1a, colp = _sc_scatter(gi[0], h[0])
    x1, x2 = _run_combine(x1a, colp)
    return (x1.reshape(2000), x2.reshape(1000))


# speedup vs baseline: 1.0138x; 1.0138x over previous
import functools

import jax
import jax.numpy as jnp
from jax import lax
from jax.experimental import pallas as pl
from jax.experimental.pallas import tpu as pltpu
from jax.experimental.pallas import tpu_sc as plsc

KPTS = 100000
NC, NS, L = 2, 16, 16
NW = NC * NS
NB = 32
CSTRIDE = 1008
NJ = CSTRIDE // L
TBL = NB * CSTRIDE
CH = 2048
NCHUNK = 49
KP = 100352
KS = CH * NCHUNK
NVREG = CH // L
TAILV = (KPTS - (NCHUNK - 1) * CH) // L
MLP_BLK = 10240
NEG = -9999.0


def _mlp_body(w1, b1, w2, b2, w3, b3, w4, b4, x_ref, r_ref, c_ref,
              h_ref, gi_ref):
    x = x_ref[...]
    h = jnp.dot(w1[...], x, preferred_element_type=jnp.float32) + b1[...]
    h = jnp.maximum(h, 0.0)
    h = jnp.dot(w2[...], h, preferred_element_type=jnp.float32) + b2[...]
    h = jnp.maximum(h, 0.0)
    h = jnp.dot(w3[...], h, preferred_element_type=jnp.float32) + b3[...]
    h = jnp.maximum(h, 0.0)
    h_ref[...] = jnp.dot(w4[...], h, preferred_element_type=jnp.float32) + b4[...]
    gi_ref[...] = r_ref[...] * CSTRIDE + c_ref[...]


def _run_mlp(x_p, rr, cc, W1, b1, W2, b2, W3, b3, W4, b4):
    full = lambda shape: pl.BlockSpec(shape, lambda i: (0, 0))
    blk = pl.BlockSpec((1, MLP_BLK), lambda i: (0, i))
    return pl.pallas_call(
        _mlp_body,
        grid=(pl.cdiv(KP, MLP_BLK),),
        in_specs=[
            full((18, 9)), full((18, 1)),
            full((36, 18)), full((36, 1)),
            full((36, 36)), full((36, 1)),
            full((1, 36)), full((1, 1)),
            pl.BlockSpec((9, MLP_BLK), lambda i: (0, i)),
            blk, blk,
        ],
        out_specs=(blk, blk),
        out_shape=(jax.ShapeDtypeStruct((1, KP), jnp.float32),
                   jax.ShapeDtypeStruct((1, KP), jnp.int32)),
    )(W1, b1.reshape(18, 1), W2, b2.reshape(36, 1), W3, b3.reshape(36, 1),
      W4, b4.reshape(1, 1), x_p, rr.reshape(1, KPTS), cc.reshape(1, KPTS))


_sc_mesh = plsc.VectorSubcoreMesh(core_axis_name="c", subcore_axis_name="s")


@functools.partial(
    pl.kernel,
    out_type=(jax.ShapeDtypeStruct((NW * NB,), jnp.float32),
              jax.ShapeDtypeStruct((NW, CSTRIDE), jnp.float32)),
    mesh=_sc_mesh,
    scratch_types=[
        pltpu.VMEM((TBL,), jnp.float32),
        pltpu.VMEM((2, CH), jnp.int32),
        pltpu.VMEM((2, CH), jnp.float32),
        pltpu.VMEM((NB,), jnp.float32),
        pltpu.VMEM((CSTRIDE,), jnp.float32),
        pltpu.VMEM_SHARED((KS,), jnp.int32),
        pltpu.VMEM_SHARED((KS,), jnp.float32),
        pltpu.SemaphoreType.DMA,
        pltpu.SemaphoreType.DMA,
        pltpu.SemaphoreType.DMA,
    ],
    compiler_params=pltpu.CompilerParams(needs_layout_passes=False),
)
def _sc_scatter(gi_hbm, h_hbm, x1a_hbm, colp_hbm,
                tbl, gb, hb, rowo, colo, gis, hs, sem0, sem1, psem):
    cid = lax.axis_index("c")
    sid = lax.axis_index("s")
    wid = sid * NC + cid
    base = wid * NB
    gilo = base * CSTRIDE
    lanes = lax.iota(jnp.int32, L)
    sems = (sem0, sem1)
    negv = jnp.full((L,), NEG, jnp.float32)

    seg = KP // NS
    soff = sid * seg
    pltpu.async_copy(gi_hbm.at[pl.ds(soff, seg)], gis.at[pl.ds(soff, seg)], psem)
    pltpu.async_copy(h_hbm.at[pl.ds(soff, seg)], hs.at[pl.ds(soff, seg)], psem)

    @plsc.parallel_loop(0, TBL // L, unroll=8)
    def _(i):
        tbl[pl.ds(i * L, L)] = negv

    pltpu.make_async_copy(gi_hbm.at[pl.ds(soff, seg)], gis.at[pl.ds(soff, seg)], psem).wait()
    pltpu.make_async_copy(h_hbm.at[pl.ds(soff, seg)], hs.at[pl.ds(soff, seg)], psem).wait()
    plsc.subcore_barrier()

    def start(slot, ci):
        off = ci * CH
        pltpu.async_copy(gis.at[pl.ds(off, CH)], gb.at[slot], sems[slot])
        pltpu.async_copy(hs.at[pl.ds(off, CH)], hb.at[slot], sems[slot])

    def wait(slot):
        pltpu.make_async_copy(gis.at[pl.ds(0, CH)], gb.at[slot], sems[slot]).wait()
        pltpu.make_async_copy(hs.at[pl.ds(0, CH)], hb.at[slot], sems[slot]).wait()

    def process(slot, nvreg, unroll):
        def vbody(i, c):
            s = i * (unroll * L)
            for j in range(unroll):
                gv = gb[slot, pl.ds(s + j * L, L)]
                hv = hb[slot, pl.ds(s + j * L, L)]
                li = gv - gilo
                m = (li >= 0) & (li < TBL)
                li = jnp.where(m, li, 0)
                plsc.store_scatter(tbl, [li], hv, mask=m)
            return c
        lax.fori_loop(0, nvreg // unroll, vbody, 0)

    start(0, 0)

    def pair_body(cj, c):
        ci0 = cj * 2
        start(1, ci0 + 1)
        wait(0)
        process(0, NVREG, 8)
        start(0, ci0 + 2)
        wait(1)
        process(1, NVREG, 8)
        return c
    lax.fori_loop(0, (NCHUNK - 1) // 2, pair_body, 0)
    wait(0)
    process(0, TAILV, 2)

    for g in range(NB // L):
        @plsc.parallel_loop(0, L, carry=negv)
        def rowvec(k, rowvec):
            rl = g * L + k
            acc = negv
            for j in range(NJ):
                acc = jnp.maximum(acc, tbl[pl.ds(rl * CSTRIDE + j * L, L)])
            rmax = jnp.max(acc)
            rmax = jnp.where(base + rl < 1000, rmax, NEG)
            return jnp.where(lanes == k, rmax, rowvec)
        rowo[pl.ds(g * L, L)] = rowvec

    nvalid = jnp.maximum(jnp.minimum(NB, 1000 - base), 0)

    for jb in range(7):
        @plsc.parallel_loop(0, nvalid, carry=(negv,) * 9)
        def accs(rl, accs):
            rbase = rl * CSTRIDE + jb * 9 * L
            return tuple(
                jnp.maximum(accs[u], tbl[pl.ds(rbase + u * L, L)])
                for u in range(9))
        for u in range(9):
            colo[pl.ds((jb * 9 + u) * L, L)] = accs[u]

    pltpu.sync_copy(rowo, x1a_hbm.at[pl.ds(base, NB)])
    pltpu.sync_copy(colo, colp_hbm.at[wid])


def _combine_body(x1a_ref, colp_ref, x1_ref, x2_ref):
    x1_ref[0, 0:NW * NB] = x1a_ref[0, :]
    x1_ref[0, NW * NB:2000] = jnp.full((2000 - NW * NB,), NEG, jnp.float32)
    red = jnp.max(colp_ref[...], axis=0)
    x2_ref[0, :] = red[0:1000]


def _run_combine(x1a, colp):
    return pl.pallas_call(
        _combine_body,
        in_specs=[pl.BlockSpec((1, NW * NB), lambda: (0, 0)),
                  pl.BlockSpec((NW, CSTRIDE), lambda: (0, 0))],
        out_specs=(pl.BlockSpec((1, 2000), lambda: (0, 0)),
                   pl.BlockSpec((1, 1000), lambda: (0, 0))),
        out_shape=(jax.ShapeDtypeStruct((1, 2000), jnp.float32),
                   jax.ShapeDtypeStruct((1, 1000), jnp.float32)),
    )(x1a.reshape(1, NW * NB), colp)


def kernel(input, T_out, T_indices, W1, b1, W2, b2, W3, b3, W4, b4):
    x = input[0, :, 0, :]
    h, gi = _run_mlp(x, T_indices[0], T_indices[1],
                     W1, b1, W2, b2, W3, b3, W4, b4)
    x1a, colp = _sc_scatter(gi[0], h[0])
    x1, x2 = _run_combine(x1a, colp)
    return (x1.reshape(2000), x2.reshape(1000))
